# baseline (speedup 1.0000x reference)
import functools

import jax
import jax.numpy as jnp
from jax import lax
from jax.experimental import pallas as pl
from jax.experimental.pallas import tpu as pltpu
from jax.experimental.pallas import tpu_sc as plsc

_H = 512
_D = 128
_NC = 2
_NS = 16
_NW = _NC * _NS
_RPW = _H // _NW
_BROWS = 528
_JC = 128
_NCH = _H // _JC
_RING = 3


def _sc_body(w_hbm, x_hbm, out_hbm, band_v, x_v, buf_v, sem):
    wid = lax.axis_index("s") * _NC + lax.axis_index("c")
    i0 = wid * _RPW
    pltpu.sync_copy(w_hbm.at[pl.ds(65520 - i0, _BROWS)], band_v)
    pltpu.sync_copy(x_hbm, x_v)

    def row_loop(ii, carry):
        for cc in range(_NCH):
            g = ii * _NCH + cc
            par = lax.rem(g, _RING)

            @pl.when(g >= _RING)
            def _():
                pltpu.make_async_copy(
                    buf_v.at[par], out_hbm.at[0, pl.ds(0, _JC)], sem).wait()

            j0 = cc * _JC
            base0 = (_RPW - 1 - ii) + j0

            @plsc.parallel_loop(0, _JC, unroll=4)
            def _(j):
                xs = plsc.load_gather(
                    x_v, [jnp.zeros((16,), jnp.int32) + (j0 + j)])
                for dc in range(_D // 16):
                    buf_v[par, j, pl.ds(dc * 16, 16)] = (
                        band_v[base0 + j, pl.ds(dc * 16, 16)] + xs)

            pltpu.make_async_copy(
                buf_v.at[par], out_hbm.at[i0 + ii, pl.ds(j0, _JC)], sem
            ).start()
        return carry

    lax.fori_loop(0, _RPW, row_loop, 0)
    for _r in range(_RING):
        pltpu.make_async_copy(
            buf_v.at[_r], out_hbm.at[0, pl.ds(0, _JC)], sem).wait()


def kernel(x, weight):
    xcol = x.reshape(_H)
    mesh = plsc.VectorSubcoreMesh(core_axis_name="c", subcore_axis_name="s")
    out3 = pl.kernel(
        _sc_body,
        out_type=jax.ShapeDtypeStruct((_H, _H, _D), jnp.float32),
        mesh=mesh,
        compiler_params=pltpu.CompilerParams(needs_layout_passes=False),
        scratch_types=[
            pltpu.VMEM((_BROWS, _D), jnp.float32),
            pltpu.VMEM((_H,), jnp.float32),
            pltpu.VMEM((_RING, _JC, _D), jnp.float32),
            pltpu.SemaphoreType.DMA,
        ],
    )(weight, xcol)
    return out3.reshape(1, _H, 1, _H, _D)


# speedup vs baseline: 1.0042x; 1.0042x over previous
import functools

import jax
import jax.numpy as jnp
from jax import lax
from jax.experimental import pallas as pl
from jax.experimental.pallas import tpu as pltpu
from jax.experimental.pallas import tpu_sc as plsc

_H = 512
_D = 128
_NC = 2
_NS = 16
_NW = _NC * _NS
_RPW = _H // _NW
_BROWS = 528
_JC = 128
_NCH = _H // _JC
_RING = 2


def _sc_body(w_hbm, x_hbm, out_hbm, band_v, x_v, buf_v, sem):
    wid = lax.axis_index("s") * _NC + lax.axis_index("c")
    i0 = wid * _RPW
    pltpu.sync_copy(w_hbm.at[pl.ds(65520 - i0, _BROWS)], band_v)
    pltpu.sync_copy(x_hbm, x_v)

    def row_loop(ii, carry):
        for cc in range(_NCH):
            par = cc % _RING
            g = ii * _NCH + cc

            @pl.when(g >= _RING)
            def _():
                pltpu.make_async_copy(
                    buf_v.at[par], out_hbm.at[0, pl.ds(0, _JC)], sem).wait()

            j0 = cc * _JC
            base0 = (_RPW - 1 - ii) + j0

            @plsc.parallel_loop(0, _JC, unroll=4)
            def _(j):
                xs = plsc.load_gather(
                    x_v, [jnp.zeros((16,), jnp.int32) + (j0 + j)])
                for dc in range(_D // 16):
                    buf_v[par, j, pl.ds(dc * 16, 16)] = (
                        band_v[base0 + j, pl.ds(dc * 16, 16)] + xs)

            pltpu.make_async_copy(
                buf_v.at[par], out_hbm.at[i0 + ii, pl.ds(j0, _JC)], sem
            ).start()
        return carry

    lax.fori_loop(0, _RPW, row_loop, 0)
    for _r in range(_RING):
        pltpu.make_async_copy(
            buf_v.at[_r], out_hbm.at[0, pl.ds(0, _JC)], sem).wait()


def kernel(x, weight):
    xcol = x.reshape(_H)
    mesh = plsc.VectorSubcoreMesh(core_axis_name="c", subcore_axis_name="s")
    out3 = pl.kernel(
        _sc_body,
        out_type=jax.ShapeDtypeStruct((_H, _H, _D), jnp.float32),
        mesh=mesh,
        compiler_params=pltpu.CompilerParams(needs_layout_passes=False),
        scratch_types=[
            pltpu.VMEM((_BROWS, _D), jnp.float32),
            pltpu.VMEM((_H,), jnp.float32),
            pltpu.VMEM((_RING, _JC, _D), jnp.float32),
            pltpu.SemaphoreType.DMA,
        ],
    )(weight, xcol)
    return out3.reshape(1, _H, 1, _H, _D)
